# baseline (speedup 1.0000x reference)
import functools

import jax
import jax.numpy as jnp
from jax import lax
from jax.experimental import pallas as pl
from jax.experimental.pallas import tpu as pltpu
from jax.experimental.pallas import tpu_sc as plsc

LAMB_A = 0.7
LAMB_B = 1.0 - 0.7

N = 50000
D = 256
NW = 32
ROWS_W = 1624
C = 56
NCH = ROWS_W // C
PIECES = D // 16


@functools.partial(
    pl.kernel,
    out_type=(
        jax.ShapeDtypeStruct((N, D), jnp.float32),
        jax.ShapeDtypeStruct((N,), jnp.int32),
    ),
    mesh=plsc.VectorSubcoreMesh(core_axis_name="c", subcore_axis_name="s"),
    scratch_types=[
        pltpu.VMEM((ROWS_W,), jnp.int32),
        pltpu.VMEM((ROWS_W,), jnp.int32),
        pltpu.VMEM((C, D), jnp.float32),
        pltpu.VMEM((C, D), jnp.float32),
        pltpu.VMEM((C, D), jnp.float32),
        pltpu.VMEM((C, D), jnp.float32),
        pltpu.VMEM((C, D), jnp.float32),
        pltpu.VMEM((C, D), jnp.float32),
        pltpu.VMEM((C, D), jnp.float32),
        pltpu.VMEM((C, D), jnp.float32),
        pltpu.SemaphoreType.DMA,
        pltpu.SemaphoreType.DMA,
        pltpu.SemaphoreType.DMA,
        pltpu.SemaphoreType.DMA,
        pltpu.SemaphoreType.DMA,
        pltpu.SemaphoreType.DMA,
        pltpu.SemaphoreType.DMA,
        pltpu.SemaphoreType.DMA,
    ],
)
def _mixup_kernel(x_hbm, y_hbm, pair_hbm, xmix_hbm, ynew_hbm,
                  idx_v, y_v, x0, x1, x2, xb0, xb1, xb2, o0, o1,
                  sx0, sx1, sx2, sg0, sg1, sg2, ss0, ss1):
    wid = lax.axis_index("s") * 2 + lax.axis_index("c")
    wbase = jnp.minimum(wid * ROWS_W, N - ROWS_W)

    x_v = (x0, x1, x2)
    xb_v = (xb0, xb1, xb2)
    o_v = (o0, o1)
    sx = (sx0, sx1, sx2)
    sg = (sg0, sg1, sg2)
    ss = (ss0, ss1)

    pltpu.sync_copy(pair_hbm.at[pl.ds(wbase, ROWS_W)], idx_v)

    def fetch(c):
        b = c % 3
        base = wbase + c * C
        dx = pltpu.async_copy(x_hbm.at[pl.ds(base, C)], x_v[b], sx[b])
        dg = pltpu.async_copy(x_hbm.at[idx_v.at[pl.ds(c * C, C)]], xb_v[b], sg[b])
        return dx, dg

    descs = {0: fetch(0), 1: fetch(1)}

    pltpu.sync_copy(y_hbm.at[pl.ds(wbase, ROWS_W)], y_v)
    pltpu.sync_copy(y_v, ynew_hbm.at[pl.ds(wbase, ROWS_W)])

    store_descs = {}
    for c in range(NCH):
        b = c % 3
        bo = c % 2
        dx, dg = descs.pop(c)
        dx.wait()
        dg.wait()
        if c + 2 < NCH:
            descs[c + 2] = fetch(c + 2)
        if c >= 2:
            store_descs[c - 2].wait()

        def row_body(i, _, b=b, bo=bo):
            for j in range(PIECES):
                sl = pl.ds(j * 16, 16)
                o_v[bo][i, sl] = LAMB_A * x_v[b][i, sl] + LAMB_B * xb_v[b][i, sl]
            return 0

        lax.fori_loop(0, C, row_body, 0, unroll=False)

        store_descs[c] = pltpu.async_copy(
            o_v[bo], xmix_hbm.at[pl.ds(wbase + c * C, C)], ss[bo])

    store_descs[NCH - 2].wait()
    store_descs[NCH - 1].wait()


def kernel(x, y, pair_idx):
    x_mix, new_y = _mixup_kernel(x, y, pair_idx)
    return (x_mix, new_y)


# speedup vs baseline: 1.0229x; 1.0229x over previous
import functools

import jax
import jax.numpy as jnp
from jax import lax
from jax.experimental import pallas as pl
from jax.experimental.pallas import tpu as pltpu
from jax.experimental.pallas import tpu_sc as plsc

LAMB_A = 0.7
LAMB_B = 1.0 - 0.7

N = 50000
D = 256
NW = 32
ROWS_W = 1568
C = 56
NCH = ROWS_W // C
PIECES = D // 16


@functools.partial(
    pl.kernel,
    out_type=(
        jax.ShapeDtypeStruct((N, D), jnp.float32),
        jax.ShapeDtypeStruct((N,), jnp.int32),
    ),
    mesh=plsc.VectorSubcoreMesh(core_axis_name="c", subcore_axis_name="s"),
    scratch_types=[
        pltpu.VMEM((ROWS_W,), jnp.int32),
        pltpu.VMEM((ROWS_W,), jnp.int32),
        pltpu.VMEM((C, D), jnp.float32),
        pltpu.VMEM((C, D), jnp.float32),
        pltpu.VMEM((C, D), jnp.float32),
        pltpu.VMEM((C, D), jnp.float32),
        pltpu.VMEM((C, D), jnp.float32),
        pltpu.VMEM((C, D), jnp.float32),
        pltpu.VMEM((C, D), jnp.float32),
        pltpu.VMEM((C, D), jnp.float32),
        pltpu.SemaphoreType.DMA,
        pltpu.SemaphoreType.DMA,
        pltpu.SemaphoreType.DMA,
        pltpu.SemaphoreType.DMA,
        pltpu.SemaphoreType.DMA,
        pltpu.SemaphoreType.DMA,
        pltpu.SemaphoreType.DMA,
        pltpu.SemaphoreType.DMA,
        pltpu.SemaphoreType.DMA,
    ],
)
def _mixup_kernel(x_hbm, y_hbm, pair_hbm, xmix_hbm, ynew_hbm,
                  idx_v, y_v, x0, x1, x2, xb0, xb1, xb2, o0, o1,
                  sx0, sx1, sx2, sg0, sg1, sg2, ss0, ss1, sy):
    wid = lax.axis_index("s") * 2 + lax.axis_index("c")
    wbase = jnp.minimum(wid * ROWS_W, N - ROWS_W)

    x_v = (x0, x1, x2)
    xb_v = (xb0, xb1, xb2)
    o_v = (o0, o1)
    sx = (sx0, sx1, sx2)
    sg = (sg0, sg1, sg2)
    ss = (ss0, ss1)

    pltpu.sync_copy(pair_hbm.at[pl.ds(wbase, ROWS_W)], idx_v)

    def fetch(c):
        b = c % 3
        base = wbase + c * C
        dx = pltpu.async_copy(x_hbm.at[pl.ds(base, C)], x_v[b], sx[b])
        dg = pltpu.async_copy(x_hbm.at[idx_v.at[pl.ds(c * C, C)]], xb_v[b], sg[b])
        return dx, dg

    descs = {0: fetch(0), 1: fetch(1)}

    dy_in = pltpu.async_copy(y_hbm.at[pl.ds(wbase, ROWS_W)], y_v, sy)
    dy_in.wait()
    dy_out = pltpu.async_copy(y_v, ynew_hbm.at[pl.ds(wbase, ROWS_W)], sy)

    store_descs = {}
    for c in range(NCH):
        b = c % 3
        bo = c % 2
        dx, dg = descs.pop(c)
        dx.wait()
        dg.wait()
        if c + 2 < NCH:
            descs[c + 2] = fetch(c + 2)
        if c >= 2:
            store_descs[c - 2].wait()

        def row_body(i, _, b=b, bo=bo):
            for j in range(PIECES):
                sl = pl.ds(j * 16, 16)
                o_v[bo][i, sl] = LAMB_A * x_v[b][i, sl] + LAMB_B * xb_v[b][i, sl]
            return 0

        lax.fori_loop(0, C, row_body, 0, unroll=False)

        store_descs[c] = pltpu.async_copy(
            o_v[bo], xmix_hbm.at[pl.ds(wbase + c * C, C)], ss[bo])

    dy_out.wait()
    store_descs[NCH - 2].wait()
    store_descs[NCH - 1].wait()


def kernel(x, y, pair_idx):
    x_mix, new_y = _mixup_kernel(x, y, pair_idx)
    return (x_mix, new_y)
